# baseline (speedup 1.0000x reference)
import functools

import jax
import jax.numpy as jnp
from jax import lax
from jax.experimental import pallas as pl
from jax.experimental.pallas import tpu as pltpu
from jax.experimental.pallas import tpu_sc as plsc

N = 10000
E0 = 320000
E = E0 + N
NC, NS, L = 2, 16, 16
NW = NC * NS
KE = 48
EPAD = 330240
PW = EPAD // NW
CHUNKS = PW // KE


def _edge_weight_kernel(D, H):
    C = D // H
    mesh = plsc.VectorSubcoreMesh(core_axis_name="c", subcore_axis_name="s")

    def body(xl_hbm, xr_hbm, src_hbm, dst_hbm, att_hbm, sh_hbm, w_hbm,
             xj, xi, sbuf, dbuf, wbuf, attbuf, shbuf, sem0, sem1):
        wid = lax.axis_index("s") * NC + lax.axis_index("c")
        base0 = wid * PW
        pltpu.sync_copy(att_hbm, attbuf)
        pltpu.sync_copy(sh_hbm, shbuf)
        lanes = lax.iota(jnp.int32, L)

        def chunk(j, carry):
            base = base0 + j * KE
            pltpu.sync_copy(src_hbm.at[pl.ds(base, KE)], sbuf)
            pltpu.sync_copy(dst_hbm.at[pl.ds(base, KE)], dbuf)
            cpj = pltpu.async_copy(xl_hbm.at[sbuf], xj, sem0)
            cpi = pltpu.async_copy(xr_hbm.at[dbuf], xi, sem1)
            cpj.wait()
            cpi.wait()
            shv = shbuf[...]
            for g in range(KE // L):
                e_lanes = g * L + lanes
                for h in range(H):
                    def cbody(cc, acc):
                        s, vs = acc
                        av = attbuf[pl.ds(cc * L, L)]
                        for k in range(L):
                            csplat = jnp.full((L,), k, jnp.int32) + cc * L
                            vj = plsc.load_gather(xj, [e_lanes, csplat])
                            vi = plsc.load_gather(xi, [e_lanes, csplat])
                            a = av[k]
                            z = vi + vj
                            lr = 0.6 * z + 0.4 * jnp.abs(z)
                            s = s + a * lr
                            vs = vs + jnp.abs(a) * jnp.abs(vi)
                        return s, vs
                    z16 = jnp.zeros((L,), jnp.float32)
                    s, vs = lax.fori_loop(
                        h * (C // L), (h + 1) * (C // L), cbody, (z16, z16))
                    wv = jnp.exp(s - vs - shv[h])
                    eid = base + e_lanes
                    wv = jnp.where(eid < E, wv, 0.0)
                    plsc.store_scatter(
                        wbuf, [e_lanes, jnp.full((L,), h, jnp.int32)], wv)
            pltpu.sync_copy(wbuf, w_hbm.at[pl.ds(base, KE)])
            return carry

        lax.fori_loop(0, CHUNKS, chunk, 0)

    return pl.kernel(
        body,
        out_type=jax.ShapeDtypeStruct((EPAD, H), jnp.float32),
        mesh=mesh,
        compiler_params=pltpu.CompilerParams(
            use_tc_tiling_on_sc=False, needs_layout_passes=False),
        scratch_types=[
            pltpu.VMEM((KE, D), jnp.float32),
            pltpu.VMEM((KE, D), jnp.float32),
            pltpu.VMEM((KE,), jnp.int32),
            pltpu.VMEM((KE,), jnp.int32),
            pltpu.VMEM((KE, H), jnp.float32),
            pltpu.VMEM((D,), jnp.float32),
            pltpu.VMEM((L,), jnp.float32),
            pltpu.SemaphoreType.DMA,
            pltpu.SemaphoreType.DMA,
        ],
    )


def _agg_l1_kernel():
    D, H, C = 512, 4, 128
    DP = D + L
    G, NG, NGR, RPT, TRASH = 8, 1250, 1264, 79, 1250
    KEL = KE + L
    mesh = plsc.VectorSubcoreMesh(core_axis_name="c", subcore_axis_name="s")

    def body(xl_hbm, src_hbm, dst_hbm, w_hbm, zeros_hbm, parts_hbm,
             sbuf, dbuf, wbuf, cs, cd, ce, xjc, scaled, shared, sem):
        cid = lax.axis_index("c")
        sid = lax.axis_index("s")
        wid = sid * NC + cid
        base0 = wid * PW
        lanes = lax.iota(jnp.int32, L)
        z16 = jnp.zeros((L,), jnp.float32)

        def zrow(i, _):
            r = i // (DP // L)
            cc = i % (DP // L)
            plsc.store_scatter(scaled, [jnp.full((L,), 0, jnp.int32) + r,
                                        cc * L + lanes], z16)
            return 0
        lax.fori_loop(0, KEL * (DP // L), zrow, 0)

        def group(g, _):
            lo = g * NG
            pltpu.sync_copy(zeros_hbm.at[pl.ds(sid * RPT, RPT)],
                            shared.at[pl.ds(sid * RPT, RPT)])
            plsc.subcore_barrier()

            def chunk(j, _2):
                base = base0 + j * KE
                pltpu.sync_copy(src_hbm.at[pl.ds(base, KE)], sbuf)
                pltpu.sync_copy(dst_hbm.at[pl.ds(base, KE)], dbuf)
                pltpu.sync_copy(w_hbm.at[pl.ds(base, KE)], wbuf)
                for q in range(KEL // L):
                    plsc.store_scatter(cd, [q * L + lanes],
                                       jnp.full((L,), TRASH, jnp.int32))
                    plsc.store_scatter(cs, [q * L + lanes],
                                       jnp.zeros((L,), jnp.int32))
                    plsc.store_scatter(ce, [q * L + lanes],
                                       jnp.zeros((L,), jnp.int32))
                cnt = 0
                for q in range(KE // L):
                    sv = sbuf[pl.ds(q * L, L)]
                    dv = dbuf[pl.ds(q * L, L)]
                    m = (dv >= lo) & (dv < lo + NG)
                    plsc.store_compressed(cs.at[pl.ds(cnt, L)], sv, mask=m)
                    plsc.store_compressed(cd.at[pl.ds(cnt, L)], dv - lo, mask=m)
                    plsc.store_compressed(ce.at[pl.ds(cnt, L)],
                                          q * L + lanes, mask=m)
                    pc = plsc.all_reduce_population_count(m)
                    cnt = cnt + pc[0]
                pltpu.async_copy(xl_hbm.at[cs], xjc, sem).wait()
                for q in range(KE // L):
                    p_lanes = q * L + lanes
                    ev = ce[pl.ds(q * L, L)]
                    for h in range(H):
                        wv = plsc.load_gather(
                            wbuf, [ev, jnp.full((L,), h, jnp.int32)])

                        def scale_cc(cc, _3):
                            for k in range(L):
                                csplat = jnp.full((L,), k, jnp.int32) + cc * L
                                vj = plsc.load_gather(xjc, [p_lanes, csplat])
                                plsc.store_scatter(scaled, [p_lanes, csplat],
                                                   vj * wv)
                            return 0
                        lax.fori_loop(h * (C // L), (h + 1) * (C // L),
                                      scale_cc, 0)
                        plsc.store_scatter(
                            scaled, [p_lanes, jnp.full((L,), D + h, jnp.int32)],
                            wv)
                pltpu.sync_copy(scaled, shared.at[cd], add=True)
                return 0
            lax.fori_loop(0, CHUNKS, chunk, 0)
            plsc.subcore_barrier()
            pltpu.sync_copy(
                shared.at[pl.ds(sid * RPT, RPT)],
                parts_hbm.at[cid, pl.ds(g * NGR + sid * RPT, RPT)])
            plsc.subcore_barrier()
            return 0
        lax.fori_loop(0, G, group, 0)

    return pl.kernel(
        body,
        out_type=jax.ShapeDtypeStruct((NC, G * NGR, DP), jnp.float32),
        mesh=mesh,
        compiler_params=pltpu.CompilerParams(
            use_tc_tiling_on_sc=False, needs_layout_passes=False),
        scratch_types=[
            pltpu.VMEM((KE,), jnp.int32),
            pltpu.VMEM((KE,), jnp.int32),
            pltpu.VMEM((KE, 4), jnp.float32),
            pltpu.VMEM((KEL,), jnp.int32),
            pltpu.VMEM((KEL,), jnp.int32),
            pltpu.VMEM((KEL,), jnp.int32),
            pltpu.VMEM((KEL, D), jnp.float32),
            pltpu.VMEM((KEL, DP), jnp.float32),
            pltpu.VMEM_SHARED((NGR, DP), jnp.float32),
            pltpu.SemaphoreType.DMA,
        ],
    )


def _layer(x, srcp, dstp, Wl, bl, Wr, br, att, bias, heads, C, concat):
    D = heads * C
    xl = x @ Wl + bl
    xr = x @ Wr + br
    aflat = jnp.abs(att).reshape(D)
    u = (jnp.abs(xl) * aflat).reshape(N, heads, C).sum(-1)
    sh = jnp.zeros((L,), jnp.float32).at[:heads].set(u.max(0))
    w = _edge_weight_kernel(D, heads)(
        xl, xr, srcp, dstp, att.reshape(D), sh)
    if heads == 4:
        G, NGR, NG, DP = 8, 1264, 1250, 528
        zeros = jnp.zeros((NGR, DP), jnp.float32)
        parts = _agg_l1_kernel()(xl, srcp, dstp, w, zeros)
        acc = (parts[0] + parts[1]).reshape(G, NGR, DP)[:, :NG, :].reshape(N, DP)
        Snum = acc[:, :D].reshape(N, heads, C)
        denom = acc[:, D:D + heads]
    else:
        w = w[:E]
        src, dst = srcp[:E], dstp[:E]
        denom = jax.ops.segment_sum(w, dst, num_segments=N)
        Snum = jax.ops.segment_sum(
            xl[src].reshape(E, heads, C) * w[..., None], dst, num_segments=N)
    out = Snum / denom[..., None]
    out = out.reshape(N, D) if concat else out.mean(1)
    return out + bias


def kernel(x, edge_index, Wl1, bl1, Wr1, br1, att1, b1, Wl2, bl2, Wr2, br2, att2, b2):
    loop = jnp.arange(N, dtype=edge_index.dtype)
    zpad = jnp.zeros((EPAD - E,), edge_index.dtype)
    srcp = jnp.concatenate([edge_index[0], loop, zpad])
    dstp = jnp.concatenate([edge_index[1], loop, zpad])
    h = jax.nn.elu(_layer(x, srcp, dstp, Wl1, bl1, Wr1, br1, att1, b1, 4, 128, True))
    z = _layer(h, srcp, dstp, Wl2, bl2, Wr2, br2, att2, b2, 1, 64, False)
    return z


# speedup vs baseline: 8.9618x; 8.9618x over previous
import jax
import jax.numpy as jnp
from jax import lax
from jax.experimental import pallas as pl
from jax.experimental.pallas import tpu as pltpu
from jax.experimental.pallas import tpu_sc as plsc

N = 10000
E0 = 320000
E = E0 + N
NC, NS, L = 2, 16, 16
NW = NC * NS
KE = 48
EPAD = 331776
PW = EPAD // NW
CHUNKS = PW // KE

_SC_PARAMS = pltpu.CompilerParams(
    use_tc_tiling_on_sc=False, needs_layout_passes=False)


def _edge_weight_kernel(D, H):
    C = D // H
    mesh = plsc.VectorSubcoreMesh(core_axis_name="c", subcore_axis_name="s")

    def body(xl_hbm, xr_hbm, src_hbm, dst_hbm, att_hbm, sh_hbm, w_hbm,
             xj0, xj1, xi0, xi1, sb0, sb1, db0, db1, wb0, wb1,
             attbuf, shbuf, sj0, sj1, si0, si1):
        xjs, xis, sbs, dbs, wbs = (xj0, xj1), (xi0, xi1), (sb0, sb1), \
            (db0, db1), (wb0, wb1)
        sjs, sis = (sj0, sj1), (si0, si1)
        wid = lax.axis_index("s") * NC + lax.axis_index("c")
        base0 = wid * PW
        pltpu.sync_copy(att_hbm, attbuf)
        pltpu.sync_copy(sh_hbm, shbuf)
        lanes = lax.iota(jnp.int32, L)

        def prefetch(j, b):
            base = base0 + j * KE
            pltpu.sync_copy(src_hbm.at[pl.ds(base, KE)], sbs[b])
            pltpu.sync_copy(dst_hbm.at[pl.ds(base, KE)], dbs[b])
            pltpu.async_copy(xl_hbm.at[sbs[b]], xjs[b], sjs[b])
            pltpu.async_copy(xr_hbm.at[dbs[b]], xis[b], sis[b])

        prefetch(0, 0)

        def loop(j2, carry):
            for b in range(2):
                j = j2 * 2 + b
                base = base0 + j * KE

                @pl.when(j + 1 < CHUNKS)
                def _():
                    prefetch(j + 1, 1 - b)
                pltpu.make_async_copy(
                    xl_hbm.at[sbs[b]], xjs[b], sjs[b]).wait()
                pltpu.make_async_copy(
                    xr_hbm.at[dbs[b]], xis[b], sis[b]).wait()
                xj, xi, wbuf = xjs[b], xis[b], wbs[b]
                shv = shbuf[...]
                for g in range(KE // L):
                    e_lanes = g * L + lanes
                    for h in range(H):
                        def cbody(cc, acc):
                            s, vs = acc
                            av = attbuf[pl.ds(cc * L, L)]
                            for k in range(L):
                                csplat = jnp.full((L,), k, jnp.int32) + cc * L
                                vj = plsc.load_gather(xj, [e_lanes, csplat])
                                vi = plsc.load_gather(xi, [e_lanes, csplat])
                                a = av[k]
                                z = vi + vj
                                lr = 0.6 * z + 0.4 * jnp.abs(z)
                                s = s + a * lr
                                vs = vs + jnp.abs(a) * jnp.abs(vi)
                            return s, vs
                        z16 = jnp.zeros((L,), jnp.float32)
                        s, vs = lax.fori_loop(
                            h * (C // L), (h + 1) * (C // L), cbody,
                            (z16, z16))
                        wv = jnp.exp(s - vs - shv[h])
                        eid = base + e_lanes
                        wv = jnp.where(eid < E, wv, 0.0)
                        plsc.store_scatter(
                            wbuf, [e_lanes, jnp.full((L,), h, jnp.int32)], wv)
                pltpu.sync_copy(wbuf, w_hbm.at[pl.ds(base, KE)])
            return carry

        lax.fori_loop(0, CHUNKS // 2, loop, 0)

    return pl.kernel(
        body,
        out_type=jax.ShapeDtypeStruct((EPAD, H), jnp.float32),
        mesh=mesh,
        compiler_params=_SC_PARAMS,
        scratch_types=[
            pltpu.VMEM((KE, D), jnp.float32),
            pltpu.VMEM((KE, D), jnp.float32),
            pltpu.VMEM((KE, D), jnp.float32),
            pltpu.VMEM((KE, D), jnp.float32),
            pltpu.VMEM((KE,), jnp.int32),
            pltpu.VMEM((KE,), jnp.int32),
            pltpu.VMEM((KE,), jnp.int32),
            pltpu.VMEM((KE,), jnp.int32),
            pltpu.VMEM((KE, H), jnp.float32),
            pltpu.VMEM((KE, H), jnp.float32),
            pltpu.VMEM((D,), jnp.float32),
            pltpu.VMEM((L,), jnp.float32),
            pltpu.SemaphoreType.DMA,
            pltpu.SemaphoreType.DMA,
            pltpu.SemaphoreType.DMA,
            pltpu.SemaphoreType.DMA,
        ],
    )


NR = 10016
RPT = NR // NS
DP = 144


def _agg_l1_kernel():
    H, C = 4, 128
    mesh = plsc.VectorSubcoreMesh(core_axis_name="c", subcore_axis_name="s")

    def body(xlh_hbm, src_hbm, dst_hbm, w_hbm, zeros_hbm, parts_hbm,
             xc0, xc1, si0, si1, db0, db1, wb0, wb1, scaled, shared,
             sg0, sg1):
        xcs, sis, dbs, wbs, sgs = (xc0, xc1), (si0, si1), (db0, db1), \
            (wb0, wb1), (sg0, sg1)
        cid = lax.axis_index("c")
        sid = lax.axis_index("s")
        wid = sid * NC + cid
        base0 = wid * PW
        lanes = lax.iota(jnp.int32, L)
        z16 = jnp.zeros((L,), jnp.float32)

        def zrow(r, _):
            plsc.store_scatter(scaled, [jnp.full((L,), 0, jnp.int32) + r,
                                        C + lanes], z16)
            return 0
        lax.fori_loop(0, KE, zrow, 0)

        def prefetch(j, b, h):
            base = base0 + j * KE
            pltpu.sync_copy(dst_hbm.at[pl.ds(base, KE)], dbs[b])
            pltpu.sync_copy(w_hbm.at[pl.ds(base, KE)], wbs[b])
            pltpu.sync_copy(src_hbm.at[pl.ds(base, KE)], sis[b])
            for q in range(KE // L):
                v = sis[b][pl.ds(q * L, L)]
                sis[b][pl.ds(q * L, L)] = v + h * N
            pltpu.async_copy(xlh_hbm.at[sis[b]], xcs[b], sgs[b])

        def head(h, _):
            pltpu.sync_copy(zeros_hbm.at[pl.ds(sid * RPT, RPT)],
                            shared.at[pl.ds(sid * RPT, RPT)])
            plsc.subcore_barrier()
            prefetch(0, 0, h)

            def loop(j2, carry):
                for b in range(2):
                    j = j2 * 2 + b

                    @pl.when(j + 1 < CHUNKS)
                    def _():
                        prefetch(j + 1, 1 - b, h)
                    pltpu.make_async_copy(
                        xlh_hbm.at[sis[b]], xcs[b], sgs[b]).wait()
                    xc, wbuf = xcs[b], wbs[b]
                    hsplat = jnp.full((L,), 0, jnp.int32) + h
                    for q in range(KE // L):
                        p_lanes = q * L + lanes
                        wv = plsc.load_gather(wbuf, [p_lanes, hsplat])

                        def scale_cc(cc, _3):
                            for k in range(L):
                                csplat = jnp.full((L,), k, jnp.int32) + cc * L
                                vj = plsc.load_gather(xc, [p_lanes, csplat])
                                plsc.store_scatter(
                                    scaled, [p_lanes, csplat], vj * wv)
                            return 0
                        lax.fori_loop(0, C // L, scale_cc, 0)
                        plsc.store_scatter(
                            scaled, [p_lanes, jnp.full((L,), C, jnp.int32)],
                            wv)
                    pltpu.sync_copy(scaled, shared.at[dbs[b]], add=True)
                return carry

            lax.fori_loop(0, CHUNKS // 2, loop, 0)
            plsc.subcore_barrier()
            pltpu.sync_copy(
                shared.at[pl.ds(sid * RPT, RPT)],
                parts_hbm.at[cid, pl.ds(h * NR + sid * RPT, RPT)])
            plsc.subcore_barrier()
            return 0

        lax.fori_loop(0, H, head, 0)

    return pl.kernel(
        body,
        out_type=jax.ShapeDtypeStruct((NC, 4 * NR, DP), jnp.float32),
        mesh=mesh,
        compiler_params=_SC_PARAMS,
        scratch_types=[
            pltpu.VMEM((KE, C), jnp.float32),
            pltpu.VMEM((KE, C), jnp.float32),
            pltpu.VMEM((KE,), jnp.int32),
            pltpu.VMEM((KE,), jnp.int32),
            pltpu.VMEM((KE,), jnp.int32),
            pltpu.VMEM((KE,), jnp.int32),
            pltpu.VMEM((KE, 4), jnp.float32),
            pltpu.VMEM((KE, 4), jnp.float32),
            pltpu.VMEM((KE, DP), jnp.float32),
            pltpu.VMEM_SHARED((NR, DP), jnp.float32),
            pltpu.SemaphoreType.DMA,
            pltpu.SemaphoreType.DMA,
        ],
    )


def _layer(x, srcp, dstp, Wl, bl, Wr, br, att, bias, heads, C, concat):
    D = heads * C
    xl = x @ Wl + bl
    xr = x @ Wr + br
    aflat = jnp.abs(att).reshape(D)
    u = (jnp.abs(xl) * aflat).reshape(N, heads, C).sum(-1)
    sh = jnp.zeros((L,), jnp.float32).at[:heads].set(u.max(0))
    w = _edge_weight_kernel(D, heads)(
        xl, xr, srcp, dstp, att.reshape(D), sh)
    if heads == 4:
        xlh = xl.reshape(N, heads, C).transpose(1, 0, 2).reshape(heads * N, C)
        zeros = jnp.zeros((NR, DP), jnp.float32)
        parts = _agg_l1_kernel()(xlh, srcp, dstp, w, zeros)
        acc = (parts[0] + parts[1]).reshape(heads, NR, DP)[:, :N, :]
        Snum = acc[:, :, :C].transpose(1, 0, 2)
        denom = acc[:, :, C].transpose(1, 0)
    else:
        w = w[:E]
        src, dst = srcp[:E], dstp[:E]
        denom = jax.ops.segment_sum(w, dst, num_segments=N)
        Snum = jax.ops.segment_sum(
            xl[src].reshape(E, heads, C) * w[..., None], dst, num_segments=N)
    out = Snum / denom[..., None]
    out = out.reshape(N, D) if concat else out.mean(1)
    return out + bias


def kernel(x, edge_index, Wl1, bl1, Wr1, br1, att1, b1, Wl2, bl2, Wr2, br2, att2, b2):
    loop = jnp.arange(N, dtype=edge_index.dtype)
    zpad = jnp.zeros((EPAD - E,), edge_index.dtype)
    srcp = jnp.concatenate([edge_index[0], loop, zpad])
    dstp = jnp.concatenate([edge_index[1], loop, zpad])
    h = jax.nn.elu(_layer(x, srcp, dstp, Wl1, bl1, Wr1, br1, att1, b1, 4, 128, True))
    z = _layer(h, srcp, dstp, Wl2, bl2, Wr2, br2, att2, b2, 1, 64, False)
    return z


# speedup vs baseline: 10.0128x; 1.1173x over previous
import jax
import jax.numpy as jnp
from jax import lax
from jax.experimental import pallas as pl
from jax.experimental.pallas import tpu as pltpu
from jax.experimental.pallas import tpu_sc as plsc

N = 10000
E0 = 320000
E = E0 + N
NC, NS, L = 2, 16, 16
NW = NC * NS
KE = 48
EPAD = 331776
PW = EPAD // NW
CHUNKS = PW // KE

_SC_PARAMS = pltpu.CompilerParams(
    use_tc_tiling_on_sc=False, needs_layout_passes=False)


def _edge_weight_kernel(D, H):
    C = D // H
    mesh = plsc.VectorSubcoreMesh(core_axis_name="c", subcore_axis_name="s")

    def body(xl_hbm, xr_hbm, src_hbm, dst_hbm, att_hbm, sh_hbm, w_hbm,
             xj0, xj1, xi0, xi1, sb0, sb1, db0, db1, wb0, wb1,
             attbuf, shbuf, sj0, sj1, si0, si1):
        xjs, xis, sbs, dbs, wbs = (xj0, xj1), (xi0, xi1), (sb0, sb1), \
            (db0, db1), (wb0, wb1)
        sjs, sis = (sj0, sj1), (si0, si1)
        wid = lax.axis_index("s") * NC + lax.axis_index("c")
        base0 = wid * PW
        pltpu.sync_copy(att_hbm, attbuf)
        pltpu.sync_copy(sh_hbm, shbuf)
        lanes = lax.iota(jnp.int32, L)

        def prefetch(j, b):
            base = base0 + j * KE
            pltpu.sync_copy(src_hbm.at[pl.ds(base, KE)], sbs[b])
            pltpu.sync_copy(dst_hbm.at[pl.ds(base, KE)], dbs[b])
            pltpu.async_copy(xl_hbm.at[sbs[b]], xjs[b], sjs[b])
            pltpu.async_copy(xr_hbm.at[dbs[b]], xis[b], sis[b])

        prefetch(0, 0)

        def loop(j2, carry):
            for b in range(2):
                j = j2 * 2 + b
                base = base0 + j * KE

                @pl.when(j + 1 < CHUNKS)
                def _():
                    prefetch(j + 1, 1 - b)
                pltpu.make_async_copy(
                    xl_hbm.at[sbs[b]], xjs[b], sjs[b]).wait()
                pltpu.make_async_copy(
                    xr_hbm.at[dbs[b]], xis[b], sis[b]).wait()
                xj, xi, wbuf = xjs[b], xis[b], wbs[b]
                shv = shbuf[...]
                for g in range(KE // L):
                    e_lanes = g * L + lanes
                    for h in range(H):
                        def cbody(cc, acc):
                            s, vs = acc
                            av = attbuf[pl.ds(cc * L, L)]
                            for k in range(L):
                                csplat = jnp.full((L,), k, jnp.int32) + cc * L
                                vj = plsc.load_gather(xj, [e_lanes, csplat])
                                vi = plsc.load_gather(xi, [e_lanes, csplat])
                                a = av[k]
                                z = vi + vj
                                lr = 0.6 * z + 0.4 * jnp.abs(z)
                                s = s + a * lr
                                vs = vs + jnp.abs(a) * jnp.abs(vi)
                            return s, vs
                        z16 = jnp.zeros((L,), jnp.float32)
                        s, vs = lax.fori_loop(
                            h * (C // L), (h + 1) * (C // L), cbody,
                            (z16, z16))
                        wv = jnp.exp(s - vs - shv[h])
                        eid = base + e_lanes
                        wv = jnp.where(eid < E, wv, 0.0)
                        plsc.store_scatter(
                            wbuf, [e_lanes, jnp.full((L,), h, jnp.int32)], wv)
                pltpu.sync_copy(wbuf, w_hbm.at[pl.ds(base, KE)])
            return carry

        lax.fori_loop(0, CHUNKS // 2, loop, 0)

    return pl.kernel(
        body,
        out_type=jax.ShapeDtypeStruct((EPAD, H), jnp.float32),
        mesh=mesh,
        compiler_params=_SC_PARAMS,
        scratch_types=[
            pltpu.VMEM((KE, D), jnp.float32),
            pltpu.VMEM((KE, D), jnp.float32),
            pltpu.VMEM((KE, D), jnp.float32),
            pltpu.VMEM((KE, D), jnp.float32),
            pltpu.VMEM((KE,), jnp.int32),
            pltpu.VMEM((KE,), jnp.int32),
            pltpu.VMEM((KE,), jnp.int32),
            pltpu.VMEM((KE,), jnp.int32),
            pltpu.VMEM((KE, H), jnp.float32),
            pltpu.VMEM((KE, H), jnp.float32),
            pltpu.VMEM((D,), jnp.float32),
            pltpu.VMEM((L,), jnp.float32),
            pltpu.SemaphoreType.DMA,
            pltpu.SemaphoreType.DMA,
            pltpu.SemaphoreType.DMA,
            pltpu.SemaphoreType.DMA,
        ],
    )


NR = 10016
RPT = NR // NS


def _agg_kernel(H, C, DP):
    mesh = plsc.VectorSubcoreMesh(core_axis_name="c", subcore_axis_name="s")

    def body(xlh_hbm, src_hbm, dst_hbm, w_hbm, zeros_hbm, parts_hbm,
             xc0, xc1, si0, si1, db0, db1, wb0, wb1, scaled, shared,
             sg0, sg1):
        xcs, sis, dbs, wbs, sgs = (xc0, xc1), (si0, si1), (db0, db1), \
            (wb0, wb1), (sg0, sg1)
        cid = lax.axis_index("c")
        sid = lax.axis_index("s")
        wid = sid * NC + cid
        base0 = wid * PW
        lanes = lax.iota(jnp.int32, L)
        z16 = jnp.zeros((L,), jnp.float32)

        def zrow(r, _):
            plsc.store_scatter(scaled, [jnp.full((L,), 0, jnp.int32) + r,
                                        C + lanes], z16)
            return 0
        lax.fori_loop(0, KE, zrow, 0)

        def prefetch(j, b, h):
            base = base0 + j * KE
            pltpu.sync_copy(dst_hbm.at[pl.ds(base, KE)], dbs[b])
            pltpu.sync_copy(w_hbm.at[pl.ds(base, KE)], wbs[b])
            pltpu.sync_copy(src_hbm.at[pl.ds(base, KE)], sis[b])
            for q in range(KE // L):
                v = sis[b][pl.ds(q * L, L)]
                sis[b][pl.ds(q * L, L)] = v + h * N
            pltpu.async_copy(xlh_hbm.at[sis[b]], xcs[b], sgs[b])

        def head(h, _):
            pltpu.sync_copy(zeros_hbm.at[pl.ds(sid * RPT, RPT)],
                            shared.at[pl.ds(sid * RPT, RPT)])
            plsc.subcore_barrier()
            prefetch(0, 0, h)

            def loop(j2, carry):
                for b in range(2):
                    j = j2 * 2 + b

                    @pl.when(j + 1 < CHUNKS)
                    def _():
                        prefetch(j + 1, 1 - b, h)
                    pltpu.make_async_copy(
                        xlh_hbm.at[sis[b]], xcs[b], sgs[b]).wait()
                    xc, wbuf = xcs[b], wbs[b]
                    hsplat = jnp.full((L,), 0, jnp.int32) + h
                    for q in range(KE // L):
                        p_lanes = q * L + lanes
                        wv = plsc.load_gather(wbuf, [p_lanes, hsplat])

                        def scale_cc(cc, _3):
                            for k in range(L):
                                csplat = jnp.full((L,), k, jnp.int32) + cc * L
                                vj = plsc.load_gather(xc, [p_lanes, csplat])
                                plsc.store_scatter(
                                    scaled, [p_lanes, csplat], vj * wv)
                            return 0
                        lax.fori_loop(0, C // L, scale_cc, 0)
                        plsc.store_scatter(
                            scaled, [p_lanes, jnp.full((L,), C, jnp.int32)],
                            wv)
                    pltpu.sync_copy(scaled, shared.at[dbs[b]], add=True)
                return carry

            lax.fori_loop(0, CHUNKS // 2, loop, 0)
            plsc.subcore_barrier()
            pltpu.sync_copy(
                shared.at[pl.ds(sid * RPT, RPT)],
                parts_hbm.at[cid, pl.ds(h * NR + sid * RPT, RPT)])
            plsc.subcore_barrier()
            return 0

        lax.fori_loop(0, H, head, 0)

    return pl.kernel(
        body,
        out_type=jax.ShapeDtypeStruct((NC, H * NR, DP), jnp.float32),
        mesh=mesh,
        compiler_params=_SC_PARAMS,
        scratch_types=[
            pltpu.VMEM((KE, C), jnp.float32),
            pltpu.VMEM((KE, C), jnp.float32),
            pltpu.VMEM((KE,), jnp.int32),
            pltpu.VMEM((KE,), jnp.int32),
            pltpu.VMEM((KE,), jnp.int32),
            pltpu.VMEM((KE,), jnp.int32),
            pltpu.VMEM((KE, H), jnp.float32),
            pltpu.VMEM((KE, H), jnp.float32),
            pltpu.VMEM((KE, DP), jnp.float32),
            pltpu.VMEM_SHARED((NR, DP), jnp.float32),
            pltpu.SemaphoreType.DMA,
            pltpu.SemaphoreType.DMA,
        ],
    )


def _pre_kernel(Din, D, H):
    C = D // H
    BN = 400

    def body(xb, Wlb, blb, Wrb, brb, ab, xlb, xrb, ub):
        xv = xb[...]
        lv = jnp.dot(xv, Wlb[...], preferred_element_type=jnp.float32) + blb[...]
        rv = jnp.dot(xv, Wrb[...], preferred_element_type=jnp.float32) + brb[...]
        xlb[...] = lv
        xrb[...] = rv
        ua = jnp.abs(lv) * ab[...]
        cols = [jnp.sum(ua[:, h * C:(h + 1) * C], axis=1, keepdims=True)
                for h in range(H)]
        ub[...] = jnp.concatenate(cols, axis=1) if H > 1 else cols[0]

    return pl.pallas_call(
        body,
        grid=(N // BN,),
        in_specs=[
            pl.BlockSpec((BN, Din), lambda i: (i, 0)),
            pl.BlockSpec((Din, D), lambda i: (0, 0)),
            pl.BlockSpec((D,), lambda i: (0,)),
            pl.BlockSpec((Din, D), lambda i: (0, 0)),
            pl.BlockSpec((D,), lambda i: (0,)),
            pl.BlockSpec((D,), lambda i: (0,)),
        ],
        out_specs=[
            pl.BlockSpec((BN, D), lambda i: (i, 0)),
            pl.BlockSpec((BN, D), lambda i: (i, 0)),
            pl.BlockSpec((BN, H), lambda i: (i, 0)),
        ],
        out_shape=[
            jax.ShapeDtypeStruct((N, D), jnp.float32),
            jax.ShapeDtypeStruct((N, D), jnp.float32),
            jax.ShapeDtypeStruct((N, H), jnp.float32),
        ],
    )


def _post_kernel(H, C, DP, elu):
    BN = 2504

    def body(pb, bb, ob):
        acc = pb[0] + pb[1]
        den = acc[:, C:C + 1]
        o = acc[:, :C] / den + bb[0]
        if elu:
            o = jnp.where(o > 0, o, jnp.exp(jnp.minimum(o, 0.0)) - 1.0)
        ob[...] = o

    return pl.pallas_call(
        body,
        grid=(NR // BN, H),
        in_specs=[
            pl.BlockSpec((NC, BN, DP), lambda i, h: (0, h * (NR // BN) + i, 0)),
            pl.BlockSpec((1, 1, C), lambda i, h: (h, 0, 0)),
        ],
        out_specs=pl.BlockSpec((BN, C), lambda i, h: (i, h)),
        out_shape=jax.ShapeDtypeStruct((NR, H * C), jnp.float32),
    )


def _layer(x, srcp, dstp, Wl, bl, Wr, br, att, bias, heads, C, concat):
    D = heads * C
    DP = C + L
    xl, xr, u = _pre_kernel(x.shape[1], D, heads)(
        x, Wl, bl, Wr, br, jnp.abs(att).reshape(D))
    sh = jnp.zeros((L,), jnp.float32).at[:heads].set(u.max(0))
    w = _edge_weight_kernel(D, heads)(
        xl, xr, srcp, dstp, att.reshape(D), sh)
    if heads > 1:
        xlh = xl.reshape(N, heads, C).transpose(1, 0, 2).reshape(heads * N, C)
    else:
        xlh = xl
    zeros = jnp.zeros((NR, DP), jnp.float32)
    parts = _agg_kernel(heads, C, DP)(xlh, srcp, dstp, w, zeros)
    out = _post_kernel(heads, C, DP, elu=concat)(
        parts, bias.reshape(heads, 1, C))[:N]
    if not concat:
        out = out.reshape(N, heads, C).mean(1)
    return out


def kernel(x, edge_index, Wl1, bl1, Wr1, br1, att1, b1, Wl2, bl2, Wr2, br2, att2, b2):
    loop = jnp.arange(N, dtype=edge_index.dtype)
    zpad = jnp.zeros((EPAD - E,), edge_index.dtype)
    srcp = jnp.concatenate([edge_index[0], loop, zpad])
    dstp = jnp.concatenate([edge_index[1], loop, zpad])
    h = _layer(x, srcp, dstp, Wl1, bl1, Wr1, br1, att1, b1, 4, 128, True)
    z = _layer(h, srcp, dstp, Wl2, bl2, Wr2, br2, att2, b2, 1, 64, False)
    return z


# speedup vs baseline: 11.8304x; 1.1815x over previous
import jax
import jax.numpy as jnp
from jax import lax
from jax.experimental import pallas as pl
from jax.experimental.pallas import tpu as pltpu
from jax.experimental.pallas import tpu_sc as plsc

N = 10000
E0 = 320000
E = E0 + N
NC, NS, L = 2, 16, 16
NW = NC * NS
KE = 48
EPAD = 331776
PW = EPAD // NW
CHUNKS = PW // KE

_SC_PARAMS = pltpu.CompilerParams(
    use_tc_tiling_on_sc=False, needs_layout_passes=False)


def _edge_weight_kernel(D, H):
    C = D // H
    D2 = D // 2
    C2 = C // 2
    mesh = plsc.VectorSubcoreMesh(core_axis_name="c", subcore_axis_name="s")

    def body(xl_hbm, xr_hbm, src_hbm, dst_hbm, a06_hbm, a04_hbm, sh_hbm,
             w_hbm, xj0, xj1, xi0, xi1, sb0, sb1, db0, db1, wb0, wb1,
             a06buf, a04buf, shbuf, sj0, sj1, si0, si1):
        xjs, xis, sbs, dbs, wbs = (xj0, xj1), (xi0, xi1), (sb0, sb1), \
            (db0, db1), (wb0, wb1)
        sjs, sis = (sj0, sj1), (si0, si1)
        wid = lax.axis_index("s") * NC + lax.axis_index("c")
        base0 = wid * PW
        pltpu.sync_copy(a06_hbm, a06buf)
        pltpu.sync_copy(a04_hbm, a04buf)
        pltpu.sync_copy(sh_hbm, shbuf)
        lanes = lax.iota(jnp.int32, L)

        def prefetch(j, b):
            base = base0 + j * KE
            pltpu.sync_copy(src_hbm.at[pl.ds(base, KE)], sbs[b])
            pltpu.sync_copy(dst_hbm.at[pl.ds(base, KE)], dbs[b])
            pltpu.async_copy(xl_hbm.at[sbs[b]], xjs[b], sjs[b])
            pltpu.async_copy(xr_hbm.at[dbs[b]], xis[b], sis[b])

        prefetch(0, 0)

        def loop(j2, carry):
            for b in range(2):
                j = j2 * 2 + b
                base = base0 + j * KE

                @pl.when(j + 1 < CHUNKS)
                def _():
                    prefetch(j + 1, 1 - b)
                pltpu.make_async_copy(
                    xl_hbm.at[sbs[b]], xjs[b], sjs[b]).wait()
                pltpu.make_async_copy(
                    xr_hbm.at[dbs[b]], xis[b], sis[b]).wait()
                xj, xi, wbuf = xjs[b], xis[b], wbs[b]
                shv = shbuf[...]
                for g in range(KE // L):
                    e_lanes = g * L + lanes
                    for h in range(H):
                        def cbody(cc, acc):
                            s, vs = acc
                            av06 = a06buf[pl.ds(cc * L, L)]
                            av04 = a04buf[pl.ds(cc * L, L)]
                            sb = jnp.zeros((2 * L,), jnp.bfloat16)
                            vb = jnp.zeros((2 * L,), jnp.bfloat16)
                            for k in range(L):
                                cw = jnp.full((L,), k, jnp.int32) + cc * L
                                a06 = plsc.bitcast(
                                    jnp.full((L,), 0, jnp.int32) + av06[k],
                                    jnp.bfloat16)
                                a04 = plsc.bitcast(
                                    jnp.full((L,), 0, jnp.int32) + av04[k],
                                    jnp.bfloat16)
                                vj = plsc.bitcast(
                                    plsc.load_gather(xj, [e_lanes, cw]),
                                    jnp.bfloat16)
                                vi = plsc.bitcast(
                                    plsc.load_gather(xi, [e_lanes, cw]),
                                    jnp.bfloat16)
                                z = vi + vj
                                sb = sb + a06 * z + a04 * jnp.abs(z)
                                vb = vb + jnp.abs(a04) * jnp.abs(vi)
                            s0, s1 = plsc.unpack(
                                sb, format=plsc.PackFormat.INTERLEAVED)
                            v0, v1 = plsc.unpack(
                                vb, format=plsc.PackFormat.INTERLEAVED)
                            return s + s0 + s1, vs + v0 + v1
                        z16 = jnp.zeros((L,), jnp.float32)
                        s, vs = lax.fori_loop(
                            h * (C2 // L), (h + 1) * (C2 // L), cbody,
                            (z16, z16))
                        wv = jnp.exp(s - 2.5 * vs - shv[h])
                        eid = base + e_lanes
                        wv = jnp.where(eid < E, wv, 0.0)
                        plsc.store_scatter(
                            wbuf, [e_lanes, jnp.full((L,), h, jnp.int32)], wv)
                pltpu.sync_copy(wbuf, w_hbm.at[pl.ds(base, KE)])
            return carry

        lax.fori_loop(0, CHUNKS // 2, loop, 0)

    return pl.kernel(
        body,
        out_type=jax.ShapeDtypeStruct((EPAD, H), jnp.float32),
        mesh=mesh,
        compiler_params=_SC_PARAMS,
        scratch_types=[
            pltpu.VMEM((KE, D2), jnp.int32),
            pltpu.VMEM((KE, D2), jnp.int32),
            pltpu.VMEM((KE, D2), jnp.int32),
            pltpu.VMEM((KE, D2), jnp.int32),
            pltpu.VMEM((KE,), jnp.int32),
            pltpu.VMEM((KE,), jnp.int32),
            pltpu.VMEM((KE,), jnp.int32),
            pltpu.VMEM((KE,), jnp.int32),
            pltpu.VMEM((KE, H), jnp.float32),
            pltpu.VMEM((KE, H), jnp.float32),
            pltpu.VMEM((D2,), jnp.int32),
            pltpu.VMEM((D2,), jnp.int32),
            pltpu.VMEM((L,), jnp.float32),
            pltpu.SemaphoreType.DMA,
            pltpu.SemaphoreType.DMA,
            pltpu.SemaphoreType.DMA,
            pltpu.SemaphoreType.DMA,
        ],
    )


NR = 10016
RPT = NR // NS


def _agg_kernel(H, C, DP):
    mesh = plsc.VectorSubcoreMesh(core_axis_name="c", subcore_axis_name="s")

    def body(xlh_hbm, src_hbm, dst_hbm, w_hbm, zeros_hbm, parts_hbm,
             xc0, xc1, si0, si1, db0, db1, wb0, wb1, scaled, shared,
             sg0, sg1):
        xcs, sis, dbs, wbs, sgs = (xc0, xc1), (si0, si1), (db0, db1), \
            (wb0, wb1), (sg0, sg1)
        cid = lax.axis_index("c")
        sid = lax.axis_index("s")
        wid = sid * NC + cid
        base0 = wid * PW
        lanes = lax.iota(jnp.int32, L)
        z16 = jnp.zeros((L,), jnp.float32)

        def zrow(r, _):
            plsc.store_scatter(scaled, [jnp.full((L,), 0, jnp.int32) + r,
                                        C + lanes], z16)
            return 0
        lax.fori_loop(0, KE, zrow, 0)

        def prefetch(j, b, h):
            base = base0 + j * KE
            pltpu.sync_copy(dst_hbm.at[pl.ds(base, KE)], dbs[b])
            pltpu.sync_copy(w_hbm.at[pl.ds(base, KE)], wbs[b])
            pltpu.sync_copy(src_hbm.at[pl.ds(base, KE)], sis[b])
            for q in range(KE // L):
                v = sis[b][pl.ds(q * L, L)]
                sis[b][pl.ds(q * L, L)] = v + h * N
            pltpu.async_copy(xlh_hbm.at[sis[b]], xcs[b], sgs[b])

        def head(h, _):
            pltpu.sync_copy(zeros_hbm.at[pl.ds(sid * RPT, RPT)],
                            shared.at[pl.ds(sid * RPT, RPT)])
            plsc.subcore_barrier()
            prefetch(0, 0, h)

            def loop(j2, carry):
                for b in range(2):
                    j = j2 * 2 + b

                    @pl.when(j + 1 < CHUNKS)
                    def _():
                        prefetch(j + 1, 1 - b, h)
                    pltpu.make_async_copy(
                        xlh_hbm.at[sis[b]], xcs[b], sgs[b]).wait()
                    xc, wbuf = xcs[b], wbs[b]
                    hsplat = jnp.full((L,), 0, jnp.int32) + h
                    for q in range(KE // L):
                        p_lanes = q * L + lanes
                        wv = plsc.load_gather(wbuf, [p_lanes, hsplat])

                        def scale_cc(cc, _3):
                            for k in range(L):
                                csplat = jnp.full((L,), k, jnp.int32) + cc * L
                                vj = plsc.load_gather(xc, [p_lanes, csplat])
                                plsc.store_scatter(
                                    scaled, [p_lanes, csplat], vj * wv)
                            return 0
                        lax.fori_loop(0, C // L, scale_cc, 0)
                        plsc.store_scatter(
                            scaled, [p_lanes, jnp.full((L,), C, jnp.int32)],
                            wv)
                    pltpu.sync_copy(scaled, shared.at[dbs[b]], add=True)
                return carry

            lax.fori_loop(0, CHUNKS // 2, loop, 0)
            plsc.subcore_barrier()
            pltpu.sync_copy(
                shared.at[pl.ds(sid * RPT, RPT)],
                parts_hbm.at[cid, pl.ds(h * NR + sid * RPT, RPT)])
            plsc.subcore_barrier()
            return 0

        lax.fori_loop(0, H, head, 0)

    return pl.kernel(
        body,
        out_type=jax.ShapeDtypeStruct((NC, H * NR, DP), jnp.float32),
        mesh=mesh,
        compiler_params=_SC_PARAMS,
        scratch_types=[
            pltpu.VMEM((KE, C), jnp.float32),
            pltpu.VMEM((KE, C), jnp.float32),
            pltpu.VMEM((KE,), jnp.int32),
            pltpu.VMEM((KE,), jnp.int32),
            pltpu.VMEM((KE,), jnp.int32),
            pltpu.VMEM((KE,), jnp.int32),
            pltpu.VMEM((KE, H), jnp.float32),
            pltpu.VMEM((KE, H), jnp.float32),
            pltpu.VMEM((KE, DP), jnp.float32),
            pltpu.VMEM_SHARED((NR, DP), jnp.float32),
            pltpu.SemaphoreType.DMA,
            pltpu.SemaphoreType.DMA,
        ],
    )


def _pre_kernel(Din, D, H):
    C = D // H
    BN = 400

    def body(xb, Wlb, blb, Wrb, brb, ab, xlb, xrb, xlqb, xrqb, ub):
        xv = xb[...]
        lv = jnp.dot(xv, Wlb[...], preferred_element_type=jnp.float32) + blb[...]
        rv = jnp.dot(xv, Wrb[...], preferred_element_type=jnp.float32) + brb[...]
        xlb[...] = lv
        xrb[...] = rv
        xlqb[...] = lv.astype(jnp.bfloat16)
        xrqb[...] = rv.astype(jnp.bfloat16)
        ua = jnp.abs(lv) * ab[...]
        cols = [jnp.sum(ua[:, h * C:(h + 1) * C], axis=1, keepdims=True)
                for h in range(H)]
        ub[...] = jnp.concatenate(cols, axis=1) if H > 1 else cols[0]

    return pl.pallas_call(
        body,
        grid=(N // BN,),
        in_specs=[
            pl.BlockSpec((BN, Din), lambda i: (i, 0)),
            pl.BlockSpec((Din, D), lambda i: (0, 0)),
            pl.BlockSpec((D,), lambda i: (0,)),
            pl.BlockSpec((Din, D), lambda i: (0, 0)),
            pl.BlockSpec((D,), lambda i: (0,)),
            pl.BlockSpec((D,), lambda i: (0,)),
        ],
        out_specs=[
            pl.BlockSpec((BN, D), lambda i: (i, 0)),
            pl.BlockSpec((BN, D), lambda i: (i, 0)),
            pl.BlockSpec((BN, D), lambda i: (i, 0)),
            pl.BlockSpec((BN, D), lambda i: (i, 0)),
            pl.BlockSpec((BN, H), lambda i: (i, 0)),
        ],
        out_shape=[
            jax.ShapeDtypeStruct((N, D), jnp.float32),
            jax.ShapeDtypeStruct((N, D), jnp.float32),
            jax.ShapeDtypeStruct((N, D), jnp.bfloat16),
            jax.ShapeDtypeStruct((N, D), jnp.bfloat16),
            jax.ShapeDtypeStruct((N, H), jnp.float32),
        ],
    )


def _post_kernel(H, C, DP, elu):
    BN = 2504

    def body(pb, bb, ob):
        acc = pb[0] + pb[1]
        den = acc[:, C:C + 1]
        o = acc[:, :C] / den + bb[0]
        if elu:
            o = jnp.where(o > 0, o, jnp.exp(jnp.minimum(o, 0.0)) - 1.0)
        ob[...] = o

    return pl.pallas_call(
        body,
        grid=(NR // BN, H),
        in_specs=[
            pl.BlockSpec((NC, BN, DP), lambda i, h: (0, h * (NR // BN) + i, 0)),
            pl.BlockSpec((1, 1, C), lambda i, h: (h, 0, 0)),
        ],
        out_specs=pl.BlockSpec((BN, C), lambda i, h: (i, h)),
        out_shape=jax.ShapeDtypeStruct((NR, H * C), jnp.float32),
    )


def _layer(x, srcp, dstp, Wl, bl, Wr, br, att, bias, heads, C, concat):
    D = heads * C
    DP = C + L
    xl, xr, xlq, xrq, u = _pre_kernel(x.shape[1], D, heads)(
        x, Wl, bl, Wr, br, jnp.abs(att).reshape(D))
    sh = jnp.zeros((L,), jnp.float32).at[:heads].set(u.max(0))

    def packpairs(v):
        return lax.bitcast_convert_type(
            v.astype(jnp.bfloat16).reshape(-1, 2), jnp.int32)

    aflat = att.reshape(D)
    xli = lax.bitcast_convert_type(xlq.reshape(N, D // 2, 2), jnp.int32)
    xri = lax.bitcast_convert_type(xrq.reshape(N, D // 2, 2), jnp.int32)
    w = _edge_weight_kernel(D, heads)(
        xli, xri, srcp, dstp, packpairs(0.6 * aflat), packpairs(0.4 * aflat),
        sh)
    if heads > 1:
        xlh = xl.reshape(N, heads, C).transpose(1, 0, 2).reshape(heads * N, C)
    else:
        xlh = xl
    zeros = jnp.zeros((NR, DP), jnp.float32)
    parts = _agg_kernel(heads, C, DP)(xlh, srcp, dstp, w, zeros)
    out = _post_kernel(heads, C, DP, elu=concat)(
        parts, bias.reshape(heads, 1, C))[:N]
    if not concat:
        out = out.reshape(N, heads, C).mean(1)
    return out


def kernel(x, edge_index, Wl1, bl1, Wr1, br1, att1, b1, Wl2, bl2, Wr2, br2, att2, b2):
    loop = jnp.arange(N, dtype=edge_index.dtype)
    zpad = jnp.zeros((EPAD - E,), edge_index.dtype)
    srcp = jnp.concatenate([edge_index[0], loop, zpad])
    dstp = jnp.concatenate([edge_index[1], loop, zpad])
    h = _layer(x, srcp, dstp, Wl1, bl1, Wr1, br1, att1, b1, 4, 128, True)
    z = _layer(h, srcp, dstp, Wl2, bl2, Wr2, br2, att2, b2, 1, 64, False)
    return z


# speedup vs baseline: 11.8845x; 1.0046x over previous
import jax
import jax.numpy as jnp
from jax import lax
from jax.experimental import pallas as pl
from jax.experimental.pallas import tpu as pltpu
from jax.experimental.pallas import tpu_sc as plsc

N = 10000
E0 = 320000
E = E0 + N
NC, NS, L = 2, 16, 16
NW = NC * NS
KE = 48
EPAD = 331776
PW = EPAD // NW
CHUNKS = PW // KE

_SC_PARAMS = pltpu.CompilerParams(
    use_tc_tiling_on_sc=False, needs_layout_passes=False)


def _edge_weight_kernel(D, H):
    C = D // H
    D2 = D // 2
    C2 = C // 2
    mesh = plsc.VectorSubcoreMesh(core_axis_name="c", subcore_axis_name="s")

    def body(xl_hbm, xr_hbm, src_hbm, dst_hbm, a06_hbm, a04_hbm, sh_hbm,
             w_hbm, xj0, xj1, xi0, xi1, sb0, sb1, db0, db1, wb0, wb1,
             a06buf, a04buf, shbuf, sj0, sj1, si0, si1):
        xjs, xis, sbs, dbs, wbs = (xj0, xj1), (xi0, xi1), (sb0, sb1), \
            (db0, db1), (wb0, wb1)
        sjs, sis = (sj0, sj1), (si0, si1)
        wid = lax.axis_index("s") * NC + lax.axis_index("c")
        base0 = wid * PW
        pltpu.sync_copy(a06_hbm, a06buf)
        pltpu.sync_copy(a04_hbm, a04buf)
        pltpu.sync_copy(sh_hbm, shbuf)
        lanes = lax.iota(jnp.int32, L)

        def prefetch(j, b):
            base = base0 + j * KE
            pltpu.sync_copy(src_hbm.at[pl.ds(base, KE)], sbs[b])
            pltpu.sync_copy(dst_hbm.at[pl.ds(base, KE)], dbs[b])
            pltpu.async_copy(xl_hbm.at[sbs[b]], xjs[b], sjs[b])
            pltpu.async_copy(xr_hbm.at[dbs[b]], xis[b], sis[b])

        prefetch(0, 0)

        def loop(j2, carry):
            for b in range(2):
                j = j2 * 2 + b
                base = base0 + j * KE

                @pl.when(j + 1 < CHUNKS)
                def _():
                    prefetch(j + 1, 1 - b)
                pltpu.make_async_copy(
                    xl_hbm.at[sbs[b]], xjs[b], sjs[b]).wait()
                pltpu.make_async_copy(
                    xr_hbm.at[dbs[b]], xis[b], sis[b]).wait()
                xj, xi, wbuf = xjs[b], xis[b], wbs[b]
                shv = shbuf[...]
                for g in range(KE // L):
                    e_lanes = g * L + lanes
                    for h in range(H):
                        def cbody(cc, acc):
                            s, vs = acc
                            av06 = a06buf[pl.ds(cc * L, L)]
                            av04 = a04buf[pl.ds(cc * L, L)]
                            for half in range(2):
                                sb = jnp.zeros((2 * L,), jnp.bfloat16)
                                vb = jnp.zeros((2 * L,), jnp.bfloat16)
                                for k in range(half * (L // 2),
                                               (half + 1) * (L // 2)):
                                    cw = jnp.full((L,), k, jnp.int32) + cc * L
                                    a06 = plsc.bitcast(
                                        jnp.full((L,), 0, jnp.int32) + av06[k],
                                        jnp.bfloat16)
                                    a04 = plsc.bitcast(
                                        jnp.full((L,), 0, jnp.int32) + av04[k],
                                        jnp.bfloat16)
                                    vj = plsc.bitcast(
                                        plsc.load_gather(xj, [e_lanes, cw]),
                                        jnp.bfloat16)
                                    vi = plsc.bitcast(
                                        plsc.load_gather(xi, [e_lanes, cw]),
                                        jnp.bfloat16)
                                    z = vi + vj
                                    sb = sb + a06 * z + a04 * jnp.abs(z)
                                    vb = vb + jnp.abs(a04) * jnp.abs(vi)
                                s0, s1 = plsc.unpack(
                                    sb, format=plsc.PackFormat.INTERLEAVED)
                                v0, v1 = plsc.unpack(
                                    vb, format=plsc.PackFormat.INTERLEAVED)
                                s = s + s0 + s1
                                vs = vs + v0 + v1
                            return s, vs
                        z16 = jnp.zeros((L,), jnp.float32)
                        s, vs = lax.fori_loop(
                            h * (C2 // L), (h + 1) * (C2 // L), cbody,
                            (z16, z16))
                        wv = jnp.exp(s - 2.5 * vs - shv[h])
                        eid = base + e_lanes
                        wv = jnp.where(eid < E, wv, 0.0)
                        plsc.store_scatter(
                            wbuf, [e_lanes, jnp.full((L,), h, jnp.int32)], wv)
                pltpu.sync_copy(wbuf, w_hbm.at[pl.ds(base, KE)])
            return carry

        lax.fori_loop(0, CHUNKS // 2, loop, 0)

    return pl.kernel(
        body,
        out_type=jax.ShapeDtypeStruct((EPAD, H), jnp.float32),
        mesh=mesh,
        compiler_params=_SC_PARAMS,
        scratch_types=[
            pltpu.VMEM((KE, D2), jnp.int32),
            pltpu.VMEM((KE, D2), jnp.int32),
            pltpu.VMEM((KE, D2), jnp.int32),
            pltpu.VMEM((KE, D2), jnp.int32),
            pltpu.VMEM((KE,), jnp.int32),
            pltpu.VMEM((KE,), jnp.int32),
            pltpu.VMEM((KE,), jnp.int32),
            pltpu.VMEM((KE,), jnp.int32),
            pltpu.VMEM((KE, H), jnp.float32),
            pltpu.VMEM((KE, H), jnp.float32),
            pltpu.VMEM((D2,), jnp.int32),
            pltpu.VMEM((D2,), jnp.int32),
            pltpu.VMEM((L,), jnp.float32),
            pltpu.SemaphoreType.DMA,
            pltpu.SemaphoreType.DMA,
            pltpu.SemaphoreType.DMA,
            pltpu.SemaphoreType.DMA,
        ],
    )


NR = 10016
RPT = NR // NS


def _agg_kernel(H, C, DP):
    mesh = plsc.VectorSubcoreMesh(core_axis_name="c", subcore_axis_name="s")

    def body(xlh_hbm, src_hbm, dst_hbm, w_hbm, zeros_hbm, parts_hbm,
             xc0, xc1, si0, si1, db0, db1, wb0, wb1, scaled, shared,
             sg0, sg1):
        xcs, sis, dbs, wbs, sgs = (xc0, xc1), (si0, si1), (db0, db1), \
            (wb0, wb1), (sg0, sg1)
        cid = lax.axis_index("c")
        sid = lax.axis_index("s")
        wid = sid * NC + cid
        base0 = wid * PW
        lanes = lax.iota(jnp.int32, L)
        z16 = jnp.zeros((L,), jnp.float32)

        def zrow(r, _):
            plsc.store_scatter(scaled, [jnp.full((L,), 0, jnp.int32) + r,
                                        C + lanes], z16)
            return 0
        lax.fori_loop(0, KE, zrow, 0)

        def prefetch(j, b, h):
            base = base0 + j * KE
            pltpu.sync_copy(dst_hbm.at[pl.ds(base, KE)], dbs[b])
            pltpu.sync_copy(w_hbm.at[pl.ds(base, KE)], wbs[b])
            pltpu.sync_copy(src_hbm.at[pl.ds(base, KE)], sis[b])
            for q in range(KE // L):
                v = sis[b][pl.ds(q * L, L)]
                sis[b][pl.ds(q * L, L)] = v + h * N
            pltpu.async_copy(xlh_hbm.at[sis[b]], xcs[b], sgs[b])

        def head(h, _):
            pltpu.sync_copy(zeros_hbm.at[pl.ds(sid * RPT, RPT)],
                            shared.at[pl.ds(sid * RPT, RPT)])
            plsc.subcore_barrier()
            prefetch(0, 0, h)

            def loop(j2, carry):
                for b in range(2):
                    j = j2 * 2 + b

                    @pl.when(j + 1 < CHUNKS)
                    def _():
                        prefetch(j + 1, 1 - b, h)
                    pltpu.make_async_copy(
                        xlh_hbm.at[sis[b]], xcs[b], sgs[b]).wait()
                    xc, wbuf = xcs[b], wbs[b]
                    hsplat = jnp.full((L,), 0, jnp.int32) + h
                    for q in range(KE // L):
                        p_lanes = q * L + lanes
                        wv = plsc.load_gather(wbuf, [p_lanes, hsplat])

                        def scale_cc(cc, _3):
                            for k in range(L):
                                csplat = jnp.full((L,), k, jnp.int32) + cc * L
                                vj = plsc.load_gather(xc, [p_lanes, csplat])
                                plsc.store_scatter(
                                    scaled, [p_lanes, csplat], vj * wv)
                            return 0
                        lax.fori_loop(0, C // L, scale_cc, 0)
                        plsc.store_scatter(
                            scaled, [p_lanes, jnp.full((L,), C, jnp.int32)],
                            wv)
                    pltpu.sync_copy(scaled, shared.at[dbs[b]], add=True)
                return carry

            lax.fori_loop(0, CHUNKS // 2, loop, 0)
            plsc.subcore_barrier()
            pltpu.sync_copy(
                shared.at[pl.ds(sid * RPT, RPT)],
                parts_hbm.at[cid, pl.ds(h * NR + sid * RPT, RPT)])
            plsc.subcore_barrier()
            return 0

        lax.fori_loop(0, H, head, 0)

    return pl.kernel(
        body,
        out_type=jax.ShapeDtypeStruct((NC, H * NR, DP), jnp.float32),
        mesh=mesh,
        compiler_params=_SC_PARAMS,
        scratch_types=[
            pltpu.VMEM((KE, C), jnp.float32),
            pltpu.VMEM((KE, C), jnp.float32),
            pltpu.VMEM((KE,), jnp.int32),
            pltpu.VMEM((KE,), jnp.int32),
            pltpu.VMEM((KE,), jnp.int32),
            pltpu.VMEM((KE,), jnp.int32),
            pltpu.VMEM((KE, H), jnp.float32),
            pltpu.VMEM((KE, H), jnp.float32),
            pltpu.VMEM((KE, DP), jnp.float32),
            pltpu.VMEM_SHARED((NR, DP), jnp.float32),
            pltpu.SemaphoreType.DMA,
            pltpu.SemaphoreType.DMA,
        ],
    )


def _pre_kernel(Din, D, H):
    C = D // H
    BN = 400

    def body(xb, Wlb, blb, Wrb, brb, ab, xlb, xrb, xlqb, xrqb, ub):
        xv = xb[...]
        lv = jnp.dot(xv, Wlb[...], preferred_element_type=jnp.float32) + blb[...]
        rv = jnp.dot(xv, Wrb[...], preferred_element_type=jnp.float32) + brb[...]
        xlb[...] = lv
        xrb[...] = rv
        xlqb[...] = lv.astype(jnp.bfloat16)
        xrqb[...] = rv.astype(jnp.bfloat16)
        ua = jnp.abs(lv) * ab[...]
        cols = [jnp.sum(ua[:, h * C:(h + 1) * C], axis=1, keepdims=True)
                for h in range(H)]
        ub[...] = jnp.concatenate(cols, axis=1) if H > 1 else cols[0]

    return pl.pallas_call(
        body,
        grid=(N // BN,),
        in_specs=[
            pl.BlockSpec((BN, Din), lambda i: (i, 0)),
            pl.BlockSpec((Din, D), lambda i: (0, 0)),
            pl.BlockSpec((D,), lambda i: (0,)),
            pl.BlockSpec((Din, D), lambda i: (0, 0)),
            pl.BlockSpec((D,), lambda i: (0,)),
            pl.BlockSpec((D,), lambda i: (0,)),
        ],
        out_specs=[
            pl.BlockSpec((BN, D), lambda i: (i, 0)),
            pl.BlockSpec((BN, D), lambda i: (i, 0)),
            pl.BlockSpec((BN, D), lambda i: (i, 0)),
            pl.BlockSpec((BN, D), lambda i: (i, 0)),
            pl.BlockSpec((BN, H), lambda i: (i, 0)),
        ],
        out_shape=[
            jax.ShapeDtypeStruct((N, D), jnp.float32),
            jax.ShapeDtypeStruct((N, D), jnp.float32),
            jax.ShapeDtypeStruct((N, D), jnp.bfloat16),
            jax.ShapeDtypeStruct((N, D), jnp.bfloat16),
            jax.ShapeDtypeStruct((N, H), jnp.float32),
        ],
    )


def _post_kernel(H, C, DP, elu):
    BN = 2504

    def body(pb, bb, ob):
        acc = pb[0] + pb[1]
        den = acc[:, C:C + 1]
        o = acc[:, :C] / den + bb[0]
        if elu:
            o = jnp.where(o > 0, o, jnp.exp(jnp.minimum(o, 0.0)) - 1.0)
        ob[...] = o

    return pl.pallas_call(
        body,
        grid=(NR // BN, H),
        in_specs=[
            pl.BlockSpec((NC, BN, DP), lambda i, h: (0, h * (NR // BN) + i, 0)),
            pl.BlockSpec((1, 1, C), lambda i, h: (h, 0, 0)),
        ],
        out_specs=pl.BlockSpec((BN, C), lambda i, h: (i, h)),
        out_shape=jax.ShapeDtypeStruct((NR, H * C), jnp.float32),
    )


def _layer(x, srcp, dstp, Wl, bl, Wr, br, att, bias, heads, C, concat):
    D = heads * C
    DP = C + L
    xl, xr, xlq, xrq, u = _pre_kernel(x.shape[1], D, heads)(
        x, Wl, bl, Wr, br, jnp.abs(att).reshape(D))
    sh = jnp.zeros((L,), jnp.float32).at[:heads].set(u.max(0))

    def packpairs(v):
        return lax.bitcast_convert_type(
            v.astype(jnp.bfloat16).reshape(-1, 2), jnp.int32)

    aflat = att.reshape(D)
    xli = lax.bitcast_convert_type(xlq.reshape(N, D // 2, 2), jnp.int32)
    xri = lax.bitcast_convert_type(xrq.reshape(N, D // 2, 2), jnp.int32)
    w = _edge_weight_kernel(D, heads)(
        xli, xri, srcp, dstp, packpairs(0.6 * aflat), packpairs(0.4 * aflat),
        sh)
    if heads > 1:
        xlh = xl.reshape(N, heads, C).transpose(1, 0, 2).reshape(heads * N, C)
    else:
        xlh = xl
    zeros = jnp.zeros((NR, DP), jnp.float32)
    parts = _agg_kernel(heads, C, DP)(xlh, srcp, dstp, w, zeros)
    out = _post_kernel(heads, C, DP, elu=concat)(
        parts, bias.reshape(heads, 1, C))[:N]
    if not concat:
        out = out.reshape(N, heads, C).mean(1)
    return out


def kernel(x, edge_index, Wl1, bl1, Wr1, br1, att1, b1, Wl2, bl2, Wr2, br2, att2, b2):
    loop = jnp.arange(N, dtype=edge_index.dtype)
    zpad = jnp.zeros((EPAD - E,), edge_index.dtype)
    srcp = jnp.concatenate([edge_index[0], loop, zpad])
    dstp = jnp.concatenate([edge_index[1], loop, zpad])
    h = _layer(x, srcp, dstp, Wl1, bl1, Wr1, br1, att1, b1, 4, 128, True)
    z = _layer(h, srcp, dstp, Wl2, bl2, Wr2, br2, att2, b2, 1, 64, False)
    return z


# speedup vs baseline: 14.2743x; 1.2011x over previous
import jax
import jax.numpy as jnp
from jax import lax
from jax.experimental import pallas as pl
from jax.experimental.pallas import tpu as pltpu
from jax.experimental.pallas import tpu_sc as plsc

N = 10000
E0 = 320000
E = E0 + N
NC, NS, L = 2, 16, 16
NW = NC * NS
KE = 48
EPAD = 331776
PW = EPAD // NW
CHUNKS = PW // KE

_SC_PARAMS = pltpu.CompilerParams(
    use_tc_tiling_on_sc=False, needs_layout_passes=False)


def _edge_weight_kernel(D, H):
    C = D // H
    D2 = D // 2
    C2 = C // 2
    mesh = plsc.VectorSubcoreMesh(core_axis_name="c", subcore_axis_name="s")

    def body(xl_hbm, xr_hbm, src_hbm, dst_hbm, a06_hbm, a04_hbm, sh_hbm,
             w_hbm, xj0, xj1, xi0, xi1, sb0, sb1, db0, db1, wb0, wb1,
             a06buf, a04buf, shbuf, sj0, sj1, si0, si1):
        xjs, xis, sbs, dbs, wbs = (xj0, xj1), (xi0, xi1), (sb0, sb1), \
            (db0, db1), (wb0, wb1)
        sjs, sis = (sj0, sj1), (si0, si1)
        wid = lax.axis_index("s") * NC + lax.axis_index("c")
        base0 = wid * PW
        pltpu.sync_copy(a06_hbm, a06buf)
        pltpu.sync_copy(a04_hbm, a04buf)
        pltpu.sync_copy(sh_hbm, shbuf)
        lanes = lax.iota(jnp.int32, L)

        def prefetch(j, b):
            base = base0 + j * KE
            pltpu.sync_copy(src_hbm.at[pl.ds(base, KE)], sbs[b])
            pltpu.sync_copy(dst_hbm.at[pl.ds(base, KE)], dbs[b])
            pltpu.async_copy(xl_hbm.at[sbs[b]], xjs[b], sjs[b])
            pltpu.async_copy(xr_hbm.at[dbs[b]], xis[b], sis[b])

        prefetch(0, 0)

        def loop(j2, carry):
            for b in range(2):
                j = j2 * 2 + b
                base = base0 + j * KE

                @pl.when(j + 1 < CHUNKS)
                def _():
                    prefetch(j + 1, 1 - b)
                pltpu.make_async_copy(
                    xl_hbm.at[sbs[b]], xjs[b], sjs[b]).wait()
                pltpu.make_async_copy(
                    xr_hbm.at[dbs[b]], xis[b], sis[b]).wait()
                xj, xi, wbuf = xjs[b], xis[b], wbs[b]
                shv = shbuf[...]
                for g in range(KE // L):
                    e_lanes = g * L + lanes
                    for h in range(H):
                        def cbody(cc, acc):
                            s, vs = acc
                            av06 = a06buf[pl.ds(cc * L, L)]
                            av04 = a04buf[pl.ds(cc * L, L)]
                            for half in range(2):
                                sb = jnp.zeros((2 * L,), jnp.bfloat16)
                                vb = jnp.zeros((2 * L,), jnp.bfloat16)
                                for k in range(half * (L // 2),
                                               (half + 1) * (L // 2)):
                                    cw = jnp.full((L,), k, jnp.int32) + cc * L
                                    a06 = plsc.bitcast(
                                        jnp.full((L,), 0, jnp.int32) + av06[k],
                                        jnp.bfloat16)
                                    a04 = plsc.bitcast(
                                        jnp.full((L,), 0, jnp.int32) + av04[k],
                                        jnp.bfloat16)
                                    vj = plsc.bitcast(
                                        plsc.load_gather(xj, [e_lanes, cw]),
                                        jnp.bfloat16)
                                    vi = plsc.bitcast(
                                        plsc.load_gather(xi, [e_lanes, cw]),
                                        jnp.bfloat16)
                                    z = vi + vj
                                    sb = sb + a06 * z + a04 * jnp.abs(z)
                                    vb = vb + jnp.abs(a04) * jnp.abs(vi)
                                s0, s1 = plsc.unpack(
                                    sb, format=plsc.PackFormat.INTERLEAVED)
                                v0, v1 = plsc.unpack(
                                    vb, format=plsc.PackFormat.INTERLEAVED)
                                s = s + s0 + s1
                                vs = vs + v0 + v1
                            return s, vs
                        z16 = jnp.zeros((L,), jnp.float32)
                        s, vs = lax.fori_loop(
                            h * (C2 // L), (h + 1) * (C2 // L), cbody,
                            (z16, z16))
                        wv = jnp.exp(s - 2.5 * vs - shv[h])
                        eid = base + e_lanes
                        wv = jnp.where(eid < E, wv, 0.0)
                        plsc.store_scatter(
                            wbuf, [e_lanes, jnp.full((L,), h, jnp.int32)], wv)
                pltpu.sync_copy(wbuf, w_hbm.at[pl.ds(base, KE)])
            return carry

        lax.fori_loop(0, CHUNKS // 2, loop, 0)

    return pl.kernel(
        body,
        out_type=jax.ShapeDtypeStruct((EPAD, H), jnp.float32),
        mesh=mesh,
        compiler_params=_SC_PARAMS,
        scratch_types=[
            pltpu.VMEM((KE, D2), jnp.int32),
            pltpu.VMEM((KE, D2), jnp.int32),
            pltpu.VMEM((KE, D2), jnp.int32),
            pltpu.VMEM((KE, D2), jnp.int32),
            pltpu.VMEM((KE,), jnp.int32),
            pltpu.VMEM((KE,), jnp.int32),
            pltpu.VMEM((KE,), jnp.int32),
            pltpu.VMEM((KE,), jnp.int32),
            pltpu.VMEM((KE, H), jnp.float32),
            pltpu.VMEM((KE, H), jnp.float32),
            pltpu.VMEM((D2,), jnp.int32),
            pltpu.VMEM((D2,), jnp.int32),
            pltpu.VMEM((L,), jnp.float32),
            pltpu.SemaphoreType.DMA,
            pltpu.SemaphoreType.DMA,
            pltpu.SemaphoreType.DMA,
            pltpu.SemaphoreType.DMA,
        ],
    )


NR = 10016
RPT = NR // NS


def _agg_kernel(H, C, DP):
    mesh = plsc.VectorSubcoreMesh(core_axis_name="c", subcore_axis_name="s")

    def body(xlh_hbm, src_hbm, dst_hbm, w_hbm, zeros_hbm, parts_hbm,
             xc0, xc1, si0, si1, db0, db1, wb0, wb1, scaled, shared,
             sg0, sg1):
        xcs, sis, dbs, wbs, sgs = (xc0, xc1), (si0, si1), (db0, db1), \
            (wb0, wb1), (sg0, sg1)
        cid = lax.axis_index("c")
        sid = lax.axis_index("s")
        wid = sid * NC + cid
        base0 = wid * PW
        lanes = lax.iota(jnp.int32, L)
        z16 = jnp.zeros((L,), jnp.float32)

        def zrow(r, _):
            plsc.store_scatter(scaled, [jnp.full((L,), 0, jnp.int32) + r,
                                        C + lanes], z16)
            return 0
        lax.fori_loop(0, KE, zrow, 0)

        def prefetch(j, b, h):
            base = base0 + j * KE
            pltpu.sync_copy(dst_hbm.at[pl.ds(base, KE)], dbs[b])
            pltpu.sync_copy(w_hbm.at[pl.ds(base, KE)], wbs[b])
            pltpu.sync_copy(src_hbm.at[pl.ds(base, KE)], sis[b])
            for q in range(KE // L):
                v = sis[b][pl.ds(q * L, L)]
                sis[b][pl.ds(q * L, L)] = v + h * N
            pltpu.async_copy(xlh_hbm.at[sis[b]], xcs[b], sgs[b])

        def head(h, _):
            pltpu.sync_copy(zeros_hbm.at[pl.ds(sid * RPT, RPT)],
                            shared.at[pl.ds(sid * RPT, RPT)])
            plsc.subcore_barrier()
            prefetch(0, 0, h)

            def loop(j2, carry):
                for b in range(2):
                    j = j2 * 2 + b

                    @pl.when(j + 1 < CHUNKS)
                    def _():
                        prefetch(j + 1, 1 - b, h)
                    pltpu.make_async_copy(
                        xlh_hbm.at[sis[b]], xcs[b], sgs[b]).wait()
                    xc, wbuf = xcs[b], wbs[b]
                    hsplat = jnp.full((L,), 0, jnp.int32) + h
                    for q in range(KE // L):
                        p_lanes = q * L + lanes
                        wv = plsc.load_gather(wbuf, [p_lanes, hsplat])

                        def scale_cc(cc, _3):
                            for k in range(L):
                                cw = jnp.full((L,), k, jnp.int32) + cc * L
                                bf = plsc.bitcast(
                                    plsc.load_gather(xc, [p_lanes, cw]),
                                    jnp.bfloat16)
                                v0, v1 = plsc.unpack(
                                    bf, format=plsc.PackFormat.INTERLEAVED)
                                c0 = jnp.full((L,), 2 * k, jnp.int32) \
                                    + cc * (2 * L)
                                plsc.store_scatter(
                                    scaled, [p_lanes, c0], v0 * wv)
                                plsc.store_scatter(
                                    scaled, [p_lanes, c0 + 1], v1 * wv)
                            return 0
                        lax.fori_loop(0, C // (2 * L), scale_cc, 0)
                        plsc.store_scatter(
                            scaled, [p_lanes, jnp.full((L,), C, jnp.int32)],
                            wv)
                    pltpu.sync_copy(scaled, shared.at[dbs[b]], add=True)
                return carry

            lax.fori_loop(0, CHUNKS // 2, loop, 0)
            plsc.subcore_barrier()
            pltpu.sync_copy(
                shared.at[pl.ds(sid * RPT, RPT)],
                parts_hbm.at[cid, pl.ds(h * NR + sid * RPT, RPT)])
            plsc.subcore_barrier()
            return 0

        lax.fori_loop(0, H, head, 0)

    return pl.kernel(
        body,
        out_type=jax.ShapeDtypeStruct((NC, H * NR, DP), jnp.float32),
        mesh=mesh,
        compiler_params=_SC_PARAMS,
        scratch_types=[
            pltpu.VMEM((KE, C // 2), jnp.int32),
            pltpu.VMEM((KE, C // 2), jnp.int32),
            pltpu.VMEM((KE,), jnp.int32),
            pltpu.VMEM((KE,), jnp.int32),
            pltpu.VMEM((KE,), jnp.int32),
            pltpu.VMEM((KE,), jnp.int32),
            pltpu.VMEM((KE, H), jnp.float32),
            pltpu.VMEM((KE, H), jnp.float32),
            pltpu.VMEM((KE, DP), jnp.float32),
            pltpu.VMEM_SHARED((NR, DP), jnp.float32),
            pltpu.SemaphoreType.DMA,
            pltpu.SemaphoreType.DMA,
        ],
    )


def _pre_kernel(Din, D, H):
    C = D // H
    BN = 400

    def body(xb, Wlb, blb, Wrb, brb, ab, xlqb, xrqb, ub):
        xv = xb[...]
        lv = jnp.dot(xv, Wlb[...], preferred_element_type=jnp.float32) + blb[...]
        rv = jnp.dot(xv, Wrb[...], preferred_element_type=jnp.float32) + brb[...]
        xlqb[...] = lv.astype(jnp.bfloat16)
        xrqb[...] = rv.astype(jnp.bfloat16)
        ua = jnp.abs(lv) * ab[...]
        cols = [jnp.sum(ua[:, h * C:(h + 1) * C], axis=1, keepdims=True)
                for h in range(H)]
        ub[...] = jnp.concatenate(cols, axis=1) if H > 1 else cols[0]

    return pl.pallas_call(
        body,
        grid=(N // BN,),
        in_specs=[
            pl.BlockSpec((BN, Din), lambda i: (i, 0)),
            pl.BlockSpec((Din, D), lambda i: (0, 0)),
            pl.BlockSpec((D,), lambda i: (0,)),
            pl.BlockSpec((Din, D), lambda i: (0, 0)),
            pl.BlockSpec((D,), lambda i: (0,)),
            pl.BlockSpec((D,), lambda i: (0,)),
        ],
        out_specs=[
            pl.BlockSpec((BN, D), lambda i: (i, 0)),
            pl.BlockSpec((BN, D), lambda i: (i, 0)),
            pl.BlockSpec((BN, H), lambda i: (i, 0)),
        ],
        out_shape=[
            jax.ShapeDtypeStruct((N, D), jnp.bfloat16),
            jax.ShapeDtypeStruct((N, D), jnp.bfloat16),
            jax.ShapeDtypeStruct((N, H), jnp.float32),
        ],
    )


def _post_kernel(H, C, DP, elu):
    BN = 2504

    def body(pb, bb, ob):
        acc = pb[0] + pb[1]
        den = acc[:, C:C + 1]
        o = acc[:, :C] / den + bb[0]
        if elu:
            o = jnp.where(o > 0, o, jnp.exp(jnp.minimum(o, 0.0)) - 1.0)
        ob[...] = o

    return pl.pallas_call(
        body,
        grid=(NR // BN, H),
        in_specs=[
            pl.BlockSpec((NC, BN, DP), lambda i, h: (0, h * (NR // BN) + i, 0)),
            pl.BlockSpec((1, 1, C), lambda i, h: (h, 0, 0)),
        ],
        out_specs=pl.BlockSpec((BN, C), lambda i, h: (i, h)),
        out_shape=jax.ShapeDtypeStruct((NR, H * C), jnp.float32),
    )


def _layer(x, srcp, dstp, Wl, bl, Wr, br, att, bias, heads, C, concat):
    D = heads * C
    DP = C + L
    xlq, xrq, u = _pre_kernel(x.shape[1], D, heads)(
        x, Wl, bl, Wr, br, jnp.abs(att).reshape(D))
    sh = jnp.zeros((L,), jnp.float32).at[:heads].set(u.max(0))

    def packpairs(v):
        return lax.bitcast_convert_type(
            v.astype(jnp.bfloat16).reshape(-1, 2), jnp.int32)

    aflat = att.reshape(D)
    xli = lax.bitcast_convert_type(xlq.reshape(N, D // 2, 2), jnp.int32)
    xri = lax.bitcast_convert_type(xrq.reshape(N, D // 2, 2), jnp.int32)
    w = _edge_weight_kernel(D, heads)(
        xli, xri, srcp, dstp, packpairs(0.6 * aflat), packpairs(0.4 * aflat),
        sh)
    if heads > 1:
        xlhq = xlq.reshape(N, heads, C).transpose(1, 0, 2).reshape(heads * N, C)
    else:
        xlhq = xlq
    xlhi = lax.bitcast_convert_type(
        xlhq.reshape(heads * N, C // 2, 2), jnp.int32)
    zeros = jnp.zeros((NR, DP), jnp.float32)
    parts = _agg_kernel(heads, C, DP)(xlhi, srcp, dstp, w, zeros)
    out = _post_kernel(heads, C, DP, elu=concat)(
        parts, bias.reshape(heads, 1, C))[:N]
    if not concat:
        out = out.reshape(N, heads, C).mean(1)
    return out


def kernel(x, edge_index, Wl1, bl1, Wr1, br1, att1, b1, Wl2, bl2, Wr2, br2, att2, b2):
    loop = jnp.arange(N, dtype=edge_index.dtype)
    zpad = jnp.zeros((EPAD - E,), edge_index.dtype)
    srcp = jnp.concatenate([edge_index[0], loop, zpad])
    dstp = jnp.concatenate([edge_index[1], loop, zpad])
    h = _layer(x, srcp, dstp, Wl1, bl1, Wr1, br1, att1, b1, 4, 128, True)
    z = _layer(h, srcp, dstp, Wl2, bl2, Wr2, br2, att2, b2, 1, 64, False)
    return z
